# Initial kernel scaffold; baseline (speedup 1.0000x reference)
import jax
import jax.numpy as jnp
from jax.experimental import pallas as pl


def kernel(edge_index, edge_label_index, emb):
    raise NotImplementedError("write your pallas kernel here")


# speedup vs baseline: 7.5984x; 7.5984x over previous
import jax
import jax.numpy as jnp
from jax import lax
from jax.experimental import pallas as pl
from jax.experimental.pallas import tpu as pltpu
from jax.experimental.pallas import tpu_sc as plsc

N = 10000
NPAD = 10240
D = 128
E = 320000
LE = 8192
NS = 16
NC = 2
CH = 128
EPT = E // NS
NCHUNKP = 160
ET = NCHUNKP * CH
NT = NPAD // NS
NTC = NT // CH
LT = LE // (NC * NS)
LCH = 128
LNCH = LT // LCH
LROWS = 8
ALPHA = 1.0 / 3.0

_BCAST_DNUMS = jax.lax.GatherDimensionNumbers(
    offset_dims=(), collapsed_slice_dims=(0,), start_index_map=(0,))


def _bcast(vec16, lane):
    idx = jnp.full((16, 1), lane, jnp.int32)
    return jax.lax.gather(vec16, idx, _BCAST_DNUMS, slice_sizes=(1,),
                          mode=jax.lax.GatherScatterMode.PROMISE_IN_BOUNDS)


def _rsqrt16(d):
    i = lax.bitcast_convert_type(d, jnp.int32)
    i = jnp.int32(0x5F3759DF) - (i >> 1)
    y = lax.bitcast_convert_type(i, jnp.float32)
    for _ in range(3):
        y = y * (1.5 - 0.5 * d * y * y)
    return y


def _sc_body(rowi, coli, srci, dsti, embf, pdots, xsh, o1h,
             acc_sh,
             rring, cring, sidx, didx,
             gbuf, nbuf, disb, dotsb,
             semg0, semg1):
    c = lax.axis_index("c")
    s = lax.axis_index("s")
    z16 = jnp.zeros((16,), jnp.float32)
    o16 = jnp.ones((16,), jnp.float32)
    nbase = s * NT
    fbase = c * NPAD

    def _fill(buf, v16):
        def _frow(r, carry):
            for q in range(D // 16):
                buf[r, pl.ds(q * 16, 16)] = v16
            return carry
        lax.fori_loop(0, CH, _frow, 0)

    _fill(nbuf, z16)
    _fill(gbuf, o16)

    def _zacc(t, carry):
        pltpu.sync_copy(nbuf, acc_sh.at[pl.ds(nbase + t * CH, CH), :])
        return carry
    lax.fori_loop(0, NTC, _zacc, 0)

    pltpu.sync_copy(srci.at[c, s], sidx)
    pltpu.sync_copy(dsti.at[c, s], didx)

    def _zdots(t, carry):
        for g in range(LCH // 16):
            dotsb[t, pl.ds(g * 16, 16)] = z16
        return carry
    lax.fori_loop(0, LROWS, _zdots, 0)
    plsc.subcore_barrier()

    def _hist(g, carry):
        pltpu.sync_copy(coli.at[s, pl.ds(g * 8, 8), :], cring)

        def _h8(k, carry2):
            pltpu.sync_copy(gbuf, acc_sh.at[cring.at[k]], add=True)
            return carry2
        lax.fori_loop(0, 8, _h8, 0)
        return carry
    lax.fori_loop(0, NCHUNKP // 8, _hist, 0)
    plsc.subcore_barrier()

    iota16 = lax.iota(jnp.int32, 16)
    zc16 = jnp.zeros((16,), jnp.int32)

    def _rs(t, carry):
        pltpu.sync_copy(acc_sh.at[pl.ds(nbase + t * CH, CH), :], nbuf)

        def _rs16(k, carry2):
            rows = k * 16 + iota16
            dv = plsc.load_gather(nbuf, [rows, zc16])
            y = _rsqrt16(dv)
            disb[pl.ds(t * CH + k * 16, 16)] = jnp.where(dv > 0.5, y, 0.0)
            return carry2
        lax.fori_loop(0, CH // 16, _rs16, 0)
        return carry
    lax.fori_loop(0, NTC, _rs, 0)

    _fill(gbuf, z16)

    def _zacc1(t, carry):
        pltpu.sync_copy(gbuf, acc_sh.at[pl.ds(nbase + t * CH, CH), :])
        return carry
    lax.fori_loop(0, NTC, _zacc1, 0)

    def _p3(t, carry):
        n0 = nbase + t * CH
        pltpu.sync_copy(embf.at[pl.ds(n0, CH), :], nbuf)

        def _grp(g, carry2):
            dv = disb[pl.ds(t * CH + g * 16, 16)]
            for j in range(16):
                r = g * 16 + j
                db = _bcast(dv, j)
                for q in range(D // 16):
                    nbuf[r, pl.ds(q * 16, 16)] = nbuf[r, pl.ds(q * 16, 16)] * db
            return carry2
        lax.fori_loop(0, CH // 16, _grp, 0)
        pltpu.sync_copy(nbuf, xsh.at[pl.ds(fbase + n0, CH), :])
        return carry
    lax.fori_loop(0, NTC, _p3, 0)
    plsc.subcore_barrier()

    def _edge_layer():
        def _grp(g, carry):
            pltpu.sync_copy(rowi.at[c, s, pl.ds(g * 8, 8), :], rring)
            pltpu.sync_copy(coli.at[s, pl.ds(g * 8, 8), :], cring)

            def _ek(k, carry2):
                pltpu.async_copy(xsh.at[rring.at[k]], gbuf, semg0).wait()
                pltpu.sync_copy(gbuf, acc_sh.at[cring.at[k]], add=True)
                return carry2
            lax.fori_loop(0, 8, _ek, 0)
            return carry
        lax.fori_loop(0, NCHUNKP // 8, _grp, 0)
        plsc.subcore_barrier()

    _edge_layer()

    def _resc1(t, carry):
        n0 = nbase + t * CH
        pltpu.sync_copy(acc_sh.at[pl.ds(n0, CH), :], nbuf)

        def _grp(g, carry2):
            dv = disb[pl.ds(t * CH + g * 16, 16)]
            for j in range(16):
                r = g * 16 + j
                db = _bcast(dv, j)
                for q in range(D // 16):
                    u = nbuf[r, pl.ds(q * 16, 16)] * db
                    nbuf[r, pl.ds(q * 16, 16)] = u * db
                    gbuf[r, pl.ds(q * 16, 16)] = u * ALPHA
            return carry2
        lax.fori_loop(0, CH // 16, _grp, 0)
        pltpu.sync_copy(nbuf, xsh.at[pl.ds(fbase + n0, CH), :])
        pltpu.sync_copy(gbuf, o1h.at[pl.ds(fbase + n0, CH), :])
        return carry
    lax.fori_loop(0, NTC, _resc1, 0)

    _fill(gbuf, z16)

    def _zacc2(t, carry):
        pltpu.sync_copy(gbuf, acc_sh.at[pl.ds(nbase + t * CH, CH), :])
        return carry
    lax.fori_loop(0, NTC, _zacc2, 0)
    plsc.subcore_barrier()

    _edge_layer()

    def _resc2(t, carry):
        n0 = nbase + t * CH
        pltpu.sync_copy(acc_sh.at[pl.ds(n0, CH), :], nbuf)

        def _sc2(g, carry2):
            dv = disb[pl.ds(t * CH + g * 16, 16)]
            for j in range(16):
                r = g * 16 + j
                db = _bcast(dv, j)
                for q in range(D // 16):
                    sl = pl.ds(q * 16, 16)
                    nbuf[r, sl] = nbuf[r, sl] * db * ALPHA
            return carry2
        lax.fori_loop(0, CH // 16, _sc2, 0)

        pltpu.sync_copy(o1h.at[pl.ds(fbase + n0, CH), :], gbuf)

        def _a1(g, carry2):
            for j in range(16):
                r = g * 16 + j
                for q in range(D // 16):
                    sl = pl.ds(q * 16, 16)
                    nbuf[r, sl] = nbuf[r, sl] + gbuf[r, sl]
            return carry2
        lax.fori_loop(0, CH // 16, _a1, 0)

        pltpu.sync_copy(embf.at[pl.ds(n0, CH), :], gbuf)

        def _a2(g, carry2):
            for j in range(16):
                r = g * 16 + j
                for q in range(D // 16):
                    sl = pl.ds(q * 16, 16)
                    nbuf[r, sl] = nbuf[r, sl] + gbuf[r, sl] * ALPHA
            return carry2
        lax.fori_loop(0, CH // 16, _a2, 0)
        pltpu.sync_copy(nbuf, xsh.at[pl.ds(fbase + n0, CH), :])
        return carry
    lax.fori_loop(0, NTC, _resc2, 0)
    plsc.subcore_barrier()

    def _lab(t, carry):
        gs = pltpu.async_copy(xsh.at[sidx.at[t]], gbuf, semg0)
        gd = pltpu.async_copy(xsh.at[didx.at[t]], nbuf, semg1)
        gs.wait()
        gd.wait()

        def _grp(g, carry2):
            rows = g * 16 + iota16
            acc = z16
            for dd in range(D):
                cd = jnp.full((16,), dd, jnp.int32)
                sv = plsc.load_gather(gbuf, [rows, cd])
                dv = plsc.load_gather(nbuf, [rows, cd])
                acc = acc + sv * dv
            dotsb[t, pl.ds(g * 16, 16)] = acc
            return carry2
        lax.fori_loop(0, LCH // 16, _grp, 0)
        return carry
    lax.fori_loop(0, LNCH, _lab, 0)
    pltpu.sync_copy(dotsb, pdots.at[c, s])


@jax.jit
def _negative_prop_sc(edge_index, edge_label_index, emb):
    ei32 = edge_index.astype(jnp.int32).reshape(2, NS, EPT)
    padv = N + jnp.arange(ET - EPT, dtype=jnp.int32) % (NPAD - N)
    padb = jnp.broadcast_to(padv, (2, NS, ET - EPT))
    eall = jnp.concatenate([ei32, padb], axis=2).reshape(2, NS, NCHUNKP, CH)
    off = jnp.array([0, NPAD], jnp.int32).reshape(NC, 1, 1, 1)
    rowi = eall[0][None] + off
    coli = eall[1]
    eli32 = edge_label_index.astype(jnp.int32).reshape(2, NC, NS, LNCH, LCH)
    lpad = jnp.full((NC, NS, LROWS - LNCH, LCH), N, jnp.int32) + off
    srci = jnp.concatenate([eli32[0] + off, lpad], axis=2)
    dsti = jnp.concatenate([eli32[1] + off, lpad], axis=2)
    embf = jnp.zeros((NPAD, D), jnp.float32).at[:N].set(emb)

    mesh = plsc.VectorSubcoreMesh(core_axis_name="c", subcore_axis_name="s")
    pdots, _xs, _o1 = pl.kernel(
        _sc_body,
        out_type=(jax.ShapeDtypeStruct((NC, NS, LROWS, LCH), jnp.float32),
                  jax.ShapeDtypeStruct((NC * NPAD, D), jnp.float32),
                  jax.ShapeDtypeStruct((NC * NPAD, D), jnp.float32)),
        mesh=mesh,
        compiler_params=pltpu.CompilerParams(needs_layout_passes=False),
        scratch_types=[
            pltpu.VMEM_SHARED((NPAD, D), jnp.float32),
            pltpu.VMEM((8, CH), jnp.int32),
            pltpu.VMEM((8, CH), jnp.int32),
            pltpu.VMEM((LROWS, LCH), jnp.int32),
            pltpu.VMEM((LROWS, LCH), jnp.int32),
            pltpu.VMEM((CH, D), jnp.float32),
            pltpu.VMEM((CH, D), jnp.float32),
            pltpu.VMEM((NT,), jnp.float32),
            pltpu.VMEM((LROWS, LCH), jnp.float32),
            pltpu.SemaphoreType.DMA,
            pltpu.SemaphoreType.DMA,
        ],
    )(rowi, coli, srci, dsti, embf)
    return pdots[:, :, :LNCH, :].reshape(LE)


def kernel(edge_index, edge_label_index, emb):
    return _negative_prop_sc(edge_index, edge_label_index, emb)


# speedup vs baseline: 9.8785x; 1.3001x over previous
import jax
import jax.numpy as jnp
from jax import lax
from jax.experimental import pallas as pl
from jax.experimental.pallas import tpu as pltpu
from jax.experimental.pallas import tpu_sc as plsc

N = 10000
NPAD = 10240
D = 128
E = 320000
LE = 8192
NS = 16
NC = 2
CH = 128
EPT = E // NS
NCHUNKP = 160
ET = NCHUNKP * CH
NT = NPAD // NS
NTC = NT // CH
LT = LE // (NC * NS)
LCH = 128
LNCH = LT // LCH
LROWS = 8
ALPHA = 1.0 / 3.0

_BCAST_DNUMS = jax.lax.GatherDimensionNumbers(
    offset_dims=(), collapsed_slice_dims=(0,), start_index_map=(0,))


def _bcast(vec16, lane):
    idx = jnp.full((16, 1), lane, jnp.int32)
    return jax.lax.gather(vec16, idx, _BCAST_DNUMS, slice_sizes=(1,),
                          mode=jax.lax.GatherScatterMode.PROMISE_IN_BOUNDS)


def _rsqrt16(d):
    i = lax.bitcast_convert_type(d, jnp.int32)
    i = jnp.int32(0x5F3759DF) - (i >> 1)
    y = lax.bitcast_convert_type(i, jnp.float32)
    for _ in range(3):
        y = y * (1.5 - 0.5 * d * y * y)
    return y


def _sc_body(rowi, coli, srci, dsti, embf, pdots, xsh, o1h,
             acc_sh,
             rring, cring, sidx, didx,
             gbuf, nbuf, disb, dotsb,
             semg0, semg1, sems0, sems1):
    c = lax.axis_index("c")
    s = lax.axis_index("s")
    z16 = jnp.zeros((16,), jnp.float32)
    o16 = jnp.ones((16,), jnp.float32)
    nbase = s * NT
    fbase = c * NPAD

    def _fill(buf, v16):
        def _frow(r, carry):
            for q in range(D // 16):
                buf[r, pl.ds(q * 16, 16)] = v16
            return carry
        lax.fori_loop(0, CH, _frow, 0)

    _fill(nbuf, z16)
    _fill(gbuf, o16)

    def _zacc(t, carry):
        pltpu.sync_copy(nbuf, acc_sh.at[pl.ds(nbase + t * CH, CH), :])
        return carry
    lax.fori_loop(0, NTC, _zacc, 0)

    pltpu.sync_copy(srci.at[c, s], sidx)
    pltpu.sync_copy(dsti.at[c, s], didx)

    def _zdots(t, carry):
        for g in range(LCH // 16):
            dotsb[t, pl.ds(g * 16, 16)] = z16
        return carry
    lax.fori_loop(0, LROWS, _zdots, 0)
    plsc.subcore_barrier()

    def _hist(g, carry):
        pltpu.sync_copy(coli.at[s, pl.ds(g * 8, 8), :], cring)

        def _h8(k, carry2):
            pltpu.async_copy(gbuf, acc_sh.at[cring.at[k]], semg0, add=True)
            return carry2
        lax.fori_loop(0, 8, _h8, 0)

        def _d8(k, carry2):
            pltpu.make_async_copy(gbuf, acc_sh.at[cring.at[k]], semg0).wait()
            return carry2
        lax.fori_loop(0, 8, _d8, 0)
        return carry
    lax.fori_loop(0, NCHUNKP // 8, _hist, 0)
    plsc.subcore_barrier()

    iota16 = lax.iota(jnp.int32, 16)
    zc16 = jnp.zeros((16,), jnp.int32)

    def _rs(t, carry):
        pltpu.sync_copy(acc_sh.at[pl.ds(nbase + t * CH, CH), :], nbuf)

        def _rs16(k, carry2):
            rows = k * 16 + iota16
            dv = plsc.load_gather(nbuf, [rows, zc16])
            y = _rsqrt16(dv)
            disb[pl.ds(t * CH + k * 16, 16)] = jnp.where(dv > 0.5, y, 0.0)
            return carry2
        lax.fori_loop(0, CH // 16, _rs16, 0)
        return carry
    lax.fori_loop(0, NTC, _rs, 0)

    _fill(gbuf, z16)

    def _zacc1(t, carry):
        pltpu.sync_copy(gbuf, acc_sh.at[pl.ds(nbase + t * CH, CH), :])
        return carry
    lax.fori_loop(0, NTC, _zacc1, 0)

    def _p3(t, carry):
        n0 = nbase + t * CH
        pltpu.sync_copy(embf.at[pl.ds(n0, CH), :], nbuf)

        def _grp(g, carry2):
            dv = disb[pl.ds(t * CH + g * 16, 16)]
            for j in range(16):
                r = g * 16 + j
                db = _bcast(dv, j)
                for q in range(D // 16):
                    nbuf[r, pl.ds(q * 16, 16)] = nbuf[r, pl.ds(q * 16, 16)] * db
            return carry2
        lax.fori_loop(0, CH // 16, _grp, 0)
        pltpu.sync_copy(nbuf, xsh.at[pl.ds(fbase + n0, CH), :])
        return carry
    lax.fori_loop(0, NTC, _p3, 0)
    plsc.subcore_barrier()

    def _edge_layer():
        bufs = (gbuf, nbuf)
        gsems = (semg0, semg1)
        ssems = (sems0, sems1)

        def _grp(g, carry):
            pltpu.sync_copy(rowi.at[c, s, pl.ds(g * 8, 8), :], rring)
            pltpu.sync_copy(coli.at[s, pl.ds(g * 8, 8), :], cring)
            pltpu.async_copy(xsh.at[rring.at[0]], gbuf, semg0)
            pltpu.async_copy(xsh.at[rring.at[1]], nbuf, semg1)
            for k in range(8):
                b = bufs[k % 2]
                pltpu.make_async_copy(
                    xsh.at[rring.at[k]], b, gsems[k % 2]).wait()
                pltpu.async_copy(
                    b, acc_sh.at[cring.at[k]], ssems[k % 2], add=True)
                if k + 2 < 8:
                    pltpu.make_async_copy(
                        b, acc_sh.at[cring.at[k]], ssems[k % 2]).wait()
                    pltpu.async_copy(
                        xsh.at[rring.at[k + 2]], b, gsems[k % 2])
            for k in (6, 7):
                pltpu.make_async_copy(
                    bufs[k % 2], acc_sh.at[cring.at[k]], ssems[k % 2]).wait()
            return carry
        lax.fori_loop(0, NCHUNKP // 8, _grp, 0)
        plsc.subcore_barrier()

    _edge_layer()

    def _resc1(t, carry):
        n0 = nbase + t * CH
        pltpu.sync_copy(acc_sh.at[pl.ds(n0, CH), :], nbuf)

        def _grp(g, carry2):
            dv = disb[pl.ds(t * CH + g * 16, 16)]
            for j in range(16):
                r = g * 16 + j
                db = _bcast(dv, j)
                for q in range(D // 16):
                    u = nbuf[r, pl.ds(q * 16, 16)] * db
                    nbuf[r, pl.ds(q * 16, 16)] = u * db
                    gbuf[r, pl.ds(q * 16, 16)] = u * ALPHA
            return carry2
        lax.fori_loop(0, CH // 16, _grp, 0)
        pltpu.sync_copy(nbuf, xsh.at[pl.ds(fbase + n0, CH), :])
        pltpu.sync_copy(gbuf, o1h.at[pl.ds(fbase + n0, CH), :])
        return carry
    lax.fori_loop(0, NTC, _resc1, 0)

    _fill(gbuf, z16)

    def _zacc2(t, carry):
        pltpu.sync_copy(gbuf, acc_sh.at[pl.ds(nbase + t * CH, CH), :])
        return carry
    lax.fori_loop(0, NTC, _zacc2, 0)
    plsc.subcore_barrier()

    _edge_layer()

    def _resc2(t, carry):
        n0 = nbase + t * CH
        pltpu.sync_copy(acc_sh.at[pl.ds(n0, CH), :], nbuf)

        def _sc2(g, carry2):
            dv = disb[pl.ds(t * CH + g * 16, 16)]
            for j in range(16):
                r = g * 16 + j
                db = _bcast(dv, j)
                for q in range(D // 16):
                    sl = pl.ds(q * 16, 16)
                    nbuf[r, sl] = nbuf[r, sl] * db * ALPHA
            return carry2
        lax.fori_loop(0, CH // 16, _sc2, 0)

        pltpu.sync_copy(o1h.at[pl.ds(fbase + n0, CH), :], gbuf)

        def _a1(g, carry2):
            for j in range(16):
                r = g * 16 + j
                for q in range(D // 16):
                    sl = pl.ds(q * 16, 16)
                    nbuf[r, sl] = nbuf[r, sl] + gbuf[r, sl]
            return carry2
        lax.fori_loop(0, CH // 16, _a1, 0)

        pltpu.sync_copy(embf.at[pl.ds(n0, CH), :], gbuf)

        def _a2(g, carry2):
            for j in range(16):
                r = g * 16 + j
                for q in range(D // 16):
                    sl = pl.ds(q * 16, 16)
                    nbuf[r, sl] = nbuf[r, sl] + gbuf[r, sl] * ALPHA
            return carry2
        lax.fori_loop(0, CH // 16, _a2, 0)
        pltpu.sync_copy(nbuf, xsh.at[pl.ds(fbase + n0, CH), :])
        return carry
    lax.fori_loop(0, NTC, _resc2, 0)
    plsc.subcore_barrier()

    def _lab(t, carry):
        gs = pltpu.async_copy(xsh.at[sidx.at[t]], gbuf, semg0)
        gd = pltpu.async_copy(xsh.at[didx.at[t]], nbuf, semg1)
        gs.wait()
        gd.wait()

        def _grp(g, carry2):
            rows = g * 16 + iota16
            acc = z16
            for dd in range(D):
                cd = jnp.full((16,), dd, jnp.int32)
                sv = plsc.load_gather(gbuf, [rows, cd])
                dv = plsc.load_gather(nbuf, [rows, cd])
                acc = acc + sv * dv
            dotsb[t, pl.ds(g * 16, 16)] = acc
            return carry2
        lax.fori_loop(0, LCH // 16, _grp, 0)
        return carry
    lax.fori_loop(0, LNCH, _lab, 0)
    pltpu.sync_copy(dotsb, pdots.at[c, s])


@jax.jit
def _negative_prop_sc(edge_index, edge_label_index, emb):
    ei32 = edge_index.astype(jnp.int32).reshape(2, NS, EPT)
    padv = N + jnp.arange(ET - EPT, dtype=jnp.int32) % (NPAD - N)
    padb = jnp.broadcast_to(padv, (2, NS, ET - EPT))
    eall = jnp.concatenate([ei32, padb], axis=2).reshape(2, NS, NCHUNKP, CH)
    off = jnp.array([0, NPAD], jnp.int32).reshape(NC, 1, 1, 1)
    rowi = eall[0][None] + off
    coli = eall[1]
    eli32 = edge_label_index.astype(jnp.int32).reshape(2, NC, NS, LNCH, LCH)
    lpad = jnp.full((NC, NS, LROWS - LNCH, LCH), N, jnp.int32) + off
    srci = jnp.concatenate([eli32[0] + off, lpad], axis=2)
    dsti = jnp.concatenate([eli32[1] + off, lpad], axis=2)
    embf = jnp.zeros((NPAD, D), jnp.float32).at[:N].set(emb)

    mesh = plsc.VectorSubcoreMesh(core_axis_name="c", subcore_axis_name="s")
    pdots, _xs, _o1 = pl.kernel(
        _sc_body,
        out_type=(jax.ShapeDtypeStruct((NC, NS, LROWS, LCH), jnp.float32),
                  jax.ShapeDtypeStruct((NC * NPAD, D), jnp.float32),
                  jax.ShapeDtypeStruct((NC * NPAD, D), jnp.float32)),
        mesh=mesh,
        compiler_params=pltpu.CompilerParams(needs_layout_passes=False),
        scratch_types=[
            pltpu.VMEM_SHARED((NPAD, D), jnp.float32),
            pltpu.VMEM((8, CH), jnp.int32),
            pltpu.VMEM((8, CH), jnp.int32),
            pltpu.VMEM((LROWS, LCH), jnp.int32),
            pltpu.VMEM((LROWS, LCH), jnp.int32),
            pltpu.VMEM((CH, D), jnp.float32),
            pltpu.VMEM((CH, D), jnp.float32),
            pltpu.VMEM((NT,), jnp.float32),
            pltpu.VMEM((LROWS, LCH), jnp.float32),
            pltpu.SemaphoreType.DMA,
            pltpu.SemaphoreType.DMA,
            pltpu.SemaphoreType.DMA,
            pltpu.SemaphoreType.DMA,
        ],
    )(rowi, coli, srci, dsti, embf)
    return pdots[:, :, :LNCH, :].reshape(LE)


def kernel(edge_index, edge_label_index, emb):
    return _negative_prop_sc(edge_index, edge_label_index, emb)


# speedup vs baseline: 14.3168x; 1.4493x over previous
import jax
import jax.numpy as jnp
from jax import lax
from jax.experimental import pallas as pl
from jax.experimental.pallas import tpu as pltpu
from jax.experimental.pallas import tpu_sc as plsc

N = 10000
NPAD = 10240
D = 128
E = 320000
LE = 8192
NS = 16
NC = 2
CH = 128
EPT = E // (NC * NS)
NCHUNK = 80
ET = NCHUNK * CH
NT = NPAD // NS
NTC = NT // CH
LT = LE // (NC * NS)
LCH = 128
LNCH = LT // LCH
LROWS = 8
ALPHA = 1.0 / 3.0

_BCAST_DNUMS = jax.lax.GatherDimensionNumbers(
    offset_dims=(), collapsed_slice_dims=(0,), start_index_map=(0,))


def _bcast(vec16, lane):
    idx = jnp.full((16, 1), lane, jnp.int32)
    return jax.lax.gather(vec16, idx, _BCAST_DNUMS, slice_sizes=(1,),
                          mode=jax.lax.GatherScatterMode.PROMISE_IN_BOUNDS)


def _rsqrt16(d):
    i = lax.bitcast_convert_type(d, jnp.int32)
    i = jnp.int32(0x5F3759DF) - (i >> 1)
    y = lax.bitcast_convert_type(i, jnp.float32)
    for _ in range(3):
        y = y * (1.5 - 0.5 * d * y * y)
    return y


def _fill(buf, v16):
    def _frow(r, carry):
        for q in range(D // 16):
            buf[r, pl.ds(q * 16, 16)] = v16
        return carry
    lax.fori_loop(0, CH, _frow, 0)


def _zero_acc_slice(acc_sh, zbuf, nbase):
    def _z(t, carry):
        pltpu.sync_copy(zbuf, acc_sh.at[pl.ds(nbase + t * CH, CH), :])
        return carry
    lax.fori_loop(0, NTC, _z, 0)


def _edge_pass(c, s, rowi, coli, xsrc, acc_sh, rring, cring, gbuf, nbuf,
               semg0, semg1, sems0, sems1):
    bufs = (gbuf, nbuf)
    gsems = (semg0, semg1)
    ssems = (sems0, sems1)

    def _grp(g, carry):
        pltpu.sync_copy(rowi.at[c, s, pl.ds(g * 8, 8), :], rring)
        pltpu.sync_copy(coli.at[c, s, pl.ds(g * 8, 8), :], cring)
        pltpu.async_copy(xsrc.at[rring.at[0]], gbuf, semg0)
        pltpu.async_copy(xsrc.at[rring.at[1]], nbuf, semg1)
        for k in range(8):
            b = bufs[k % 2]
            pltpu.make_async_copy(
                xsrc.at[rring.at[k]], b, gsems[k % 2]).wait()
            pltpu.async_copy(
                b, acc_sh.at[cring.at[k]], ssems[k % 2], add=True)
            if k + 2 < 8:
                pltpu.make_async_copy(
                    b, acc_sh.at[cring.at[k]], ssems[k % 2]).wait()
                pltpu.async_copy(
                    xsrc.at[rring.at[k + 2]], b, gsems[k % 2])
        for k in (6, 7):
            pltpu.make_async_copy(
                bufs[k % 2], acc_sh.at[cring.at[k]], ssems[k % 2]).wait()
        return carry
    lax.fori_loop(0, NCHUNK // 8, _grp, 0)
    plsc.subcore_barrier()


def _k1_body(coli, pdeg, acc_sh, cring, gbuf, nbuf, pbuf, semg0):
    c = lax.axis_index("c")
    s = lax.axis_index("s")
    nbase = s * NT
    z16 = jnp.zeros((16,), jnp.float32)
    o16 = jnp.ones((16,), jnp.float32)
    _fill(nbuf, z16)
    _fill(gbuf, o16)
    _zero_acc_slice(acc_sh, nbuf, nbase)
    plsc.subcore_barrier()

    def _hist(g, carry):
        pltpu.sync_copy(coli.at[c, s, pl.ds(g * 8, 8), :], cring)

        def _h8(k, carry2):
            pltpu.async_copy(gbuf, acc_sh.at[cring.at[k]], semg0, add=True)
            return carry2
        lax.fori_loop(0, 8, _h8, 0)

        def _d8(k, carry2):
            pltpu.make_async_copy(gbuf, acc_sh.at[cring.at[k]], semg0).wait()
            return carry2
        lax.fori_loop(0, 8, _d8, 0)
        return carry
    lax.fori_loop(0, NCHUNK // 8, _hist, 0)
    plsc.subcore_barrier()

    iota16 = lax.iota(jnp.int32, 16)
    zc16 = jnp.zeros((16,), jnp.int32)

    def _ext(t, carry):
        pltpu.sync_copy(acc_sh.at[pl.ds(nbase + t * CH, CH), :], nbuf)

        def _e16(k, carry2):
            rows = k * 16 + iota16
            dv = plsc.load_gather(nbuf, [rows, zc16])
            pbuf[t, pl.ds(k * 16, 16)] = dv
            return carry2
        lax.fori_loop(0, CH // 16, _e16, 0)
        return carry
    lax.fori_loop(0, NTC, _ext, 0)
    pltpu.sync_copy(pbuf, pdeg.at[c, s])


def _k2_body(rowi, coli, pdeg, embf, p1, dish, xsh,
             acc_sh, rring, cring, gbuf, nbuf, dbuf, ebuf,
             semg0, semg1, sems0, sems1):
    c = lax.axis_index("c")
    s = lax.axis_index("s")
    nbase = s * NT
    fbase = c * NPAD
    z16 = jnp.zeros((16,), jnp.float32)
    _fill(nbuf, z16)
    _zero_acc_slice(acc_sh, nbuf, nbase)

    pltpu.sync_copy(pdeg.at[0, s], dbuf)
    pltpu.sync_copy(pdeg.at[1, s], ebuf)

    def _rs(t, carry):
        for k in range(CH // 16):
            sl = pl.ds(k * 16, 16)
            dv = dbuf[t, sl] + ebuf[t, sl]
            y = _rsqrt16(dv)
            dbuf[t, sl] = jnp.where(dv > 0.5, y, 0.0)
        return carry
    lax.fori_loop(0, NTC, _rs, 0)
    pltpu.sync_copy(dbuf, dish.at[c, s])

    def _p3(t, carry):
        n0 = nbase + t * CH
        pltpu.sync_copy(embf.at[pl.ds(n0, CH), :], nbuf)

        def _grp(g, carry2):
            dv = dbuf[t, pl.ds(g * 16, 16)]
            for j in range(16):
                r = g * 16 + j
                db = _bcast(dv, j)
                for q in range(D // 16):
                    sl = pl.ds(q * 16, 16)
                    nbuf[r, sl] = nbuf[r, sl] * db
            return carry2
        lax.fori_loop(0, CH // 16, _grp, 0)
        pltpu.sync_copy(nbuf, xsh.at[pl.ds(fbase + n0, CH), :])
        return carry
    lax.fori_loop(0, NTC, _p3, 0)
    plsc.subcore_barrier()

    _edge_pass(c, s, rowi, coli, xsh, acc_sh, rring, cring, gbuf, nbuf,
               semg0, semg1, sems0, sems1)

    def _wr(t, carry):
        n0 = nbase + t * CH
        pltpu.sync_copy(acc_sh.at[pl.ds(n0, CH), :], nbuf)
        pltpu.sync_copy(nbuf, p1.at[c, pl.ds(n0, CH), :])
        return carry
    lax.fori_loop(0, NTC, _wr, 0)


def _k3_body(rowi, coli, p1, dish, p2, o1h, xs2h,
             acc_sh, rring, cring, gbuf, nbuf, dbuf,
             semg0, semg1, sems0, sems1):
    c = lax.axis_index("c")
    s = lax.axis_index("s")
    nbase = s * NT
    fbase = c * NPAD
    z16 = jnp.zeros((16,), jnp.float32)
    _fill(nbuf, z16)
    _zero_acc_slice(acc_sh, nbuf, nbase)
    pltpu.sync_copy(dish.at[c, s], dbuf)

    def _mid(t, carry):
        n0 = nbase + t * CH
        pltpu.sync_copy(p1.at[0, pl.ds(n0, CH), :], nbuf)
        pltpu.sync_copy(p1.at[1, pl.ds(n0, CH), :], gbuf)

        def _grp(g, carry2):
            dv = dbuf[t, pl.ds(g * 16, 16)]
            for j in range(16):
                r = g * 16 + j
                db = _bcast(dv, j)
                for q in range(D // 16):
                    sl = pl.ds(q * 16, 16)
                    u = (nbuf[r, sl] + gbuf[r, sl]) * db
                    nbuf[r, sl] = u * db
                    gbuf[r, sl] = u * ALPHA
            return carry2
        lax.fori_loop(0, CH // 16, _grp, 0)
        pltpu.sync_copy(nbuf, xs2h.at[pl.ds(fbase + n0, CH), :])
        pltpu.sync_copy(gbuf, o1h.at[pl.ds(fbase + n0, CH), :])
        return carry
    lax.fori_loop(0, NTC, _mid, 0)
    plsc.subcore_barrier()

    _edge_pass(c, s, rowi, coli, xs2h, acc_sh, rring, cring, gbuf, nbuf,
               semg0, semg1, sems0, sems1)

    def _wr(t, carry):
        n0 = nbase + t * CH
        pltpu.sync_copy(acc_sh.at[pl.ds(n0, CH), :], nbuf)
        pltpu.sync_copy(nbuf, p2.at[c, pl.ds(n0, CH), :])
        return carry
    lax.fori_loop(0, NTC, _wr, 0)


def _k4_body(srci, dsti, p2, dish, o1h, embf, pdots, outh,
             sidx, didx, gbuf, nbuf, dbuf, dotsb, semg0, semg1):
    c = lax.axis_index("c")
    s = lax.axis_index("s")
    nbase = s * NT
    fbase = c * NPAD
    z16 = jnp.zeros((16,), jnp.float32)
    pltpu.sync_copy(dish.at[c, s], dbuf)
    pltpu.sync_copy(srci.at[c, s], sidx)
    pltpu.sync_copy(dsti.at[c, s], didx)

    def _zdots(t, carry):
        for g in range(LCH // 16):
            dotsb[t, pl.ds(g * 16, 16)] = z16
        return carry
    lax.fori_loop(0, LROWS, _zdots, 0)

    def _out(t, carry):
        n0 = nbase + t * CH
        pltpu.sync_copy(p2.at[0, pl.ds(n0, CH), :], nbuf)
        pltpu.sync_copy(p2.at[1, pl.ds(n0, CH), :], gbuf)

        def _g1(g, carry2):
            dv = dbuf[t, pl.ds(g * 16, 16)]
            for j in range(16):
                r = g * 16 + j
                db = _bcast(dv, j)
                for q in range(D // 16):
                    sl = pl.ds(q * 16, 16)
                    nbuf[r, sl] = (nbuf[r, sl] + gbuf[r, sl]) * db * ALPHA
            return carry2
        lax.fori_loop(0, CH // 16, _g1, 0)

        pltpu.sync_copy(o1h.at[pl.ds(fbase + n0, CH), :], gbuf)

        def _g2(g, carry2):
            for j in range(16):
                r = g * 16 + j
                for q in range(D // 16):
                    sl = pl.ds(q * 16, 16)
                    nbuf[r, sl] = nbuf[r, sl] + gbuf[r, sl]
            return carry2
        lax.fori_loop(0, CH // 16, _g2, 0)

        pltpu.sync_copy(embf.at[pl.ds(n0, CH), :], gbuf)

        def _g3(g, carry2):
            for j in range(16):
                r = g * 16 + j
                for q in range(D // 16):
                    sl = pl.ds(q * 16, 16)
                    nbuf[r, sl] = nbuf[r, sl] + gbuf[r, sl] * ALPHA
            return carry2
        lax.fori_loop(0, CH // 16, _g3, 0)
        pltpu.sync_copy(nbuf, outh.at[pl.ds(fbase + n0, CH), :])
        return carry
    lax.fori_loop(0, NTC, _out, 0)
    plsc.subcore_barrier()

    iota16 = lax.iota(jnp.int32, 16)

    def _lab(t, carry):
        gs = pltpu.async_copy(outh.at[sidx.at[t]], gbuf, semg0)
        gd = pltpu.async_copy(outh.at[didx.at[t]], nbuf, semg1)
        gs.wait()
        gd.wait()

        def _grp(g, carry2):
            rows = g * 16 + iota16
            acc = z16
            for dd in range(D):
                cd = jnp.full((16,), dd, jnp.int32)
                sv = plsc.load_gather(gbuf, [rows, cd])
                dv = plsc.load_gather(nbuf, [rows, cd])
                acc = acc + sv * dv
            dotsb[t, pl.ds(g * 16, 16)] = acc
            return carry2
        lax.fori_loop(0, LCH // 16, _grp, 0)
        return carry
    lax.fori_loop(0, LNCH, _lab, 0)
    pltpu.sync_copy(dotsb, pdots.at[c, s])


@jax.jit
def _negative_prop_sc(edge_index, edge_label_index, emb):
    ei32 = edge_index.astype(jnp.int32).reshape(2, NC, NS, EPT)
    padv = N + jnp.arange(ET - EPT, dtype=jnp.int32) % (NPAD - N)
    padb = jnp.broadcast_to(padv, (2, NC, NS, ET - EPT))
    eall = jnp.concatenate([ei32, padb], axis=3).reshape(
        2, NC, NS, NCHUNK, CH)
    off = jnp.array([0, NPAD], jnp.int32).reshape(NC, 1, 1, 1)
    rowi = eall[0] + off
    coli = eall[1]
    eli32 = edge_label_index.astype(jnp.int32).reshape(2, NC, NS, LNCH, LCH)
    loff = jnp.array([0, NPAD], jnp.int32).reshape(NC, 1, 1, 1)
    lpad = jnp.full((NC, NS, LROWS - LNCH, LCH), N, jnp.int32) + loff
    srci = jnp.concatenate([eli32[0] + loff, lpad], axis=2)
    dsti = jnp.concatenate([eli32[1] + loff, lpad], axis=2)
    embf = jnp.zeros((NPAD, D), jnp.float32).at[:N].set(emb)

    mesh = plsc.VectorSubcoreMesh(core_axis_name="c", subcore_axis_name="s")
    cparams = pltpu.CompilerParams(needs_layout_passes=False)
    f32 = jnp.float32

    pdeg = pl.kernel(
        _k1_body,
        out_type=jax.ShapeDtypeStruct((NC, NS, 8, CH), f32),
        mesh=mesh, compiler_params=cparams,
        scratch_types=[
            pltpu.VMEM_SHARED((NPAD, D), f32),
            pltpu.VMEM((8, CH), jnp.int32),
            pltpu.VMEM((CH, D), f32),
            pltpu.VMEM((CH, D), f32),
            pltpu.VMEM((8, CH), f32),
            pltpu.SemaphoreType.DMA,
        ],
    )(coli)

    p1, dish, _xs = pl.kernel(
        _k2_body,
        out_type=(jax.ShapeDtypeStruct((NC, NPAD, D), f32),
                  jax.ShapeDtypeStruct((NC, NS, 8, CH), f32),
                  jax.ShapeDtypeStruct((NC * NPAD, D), f32)),
        mesh=mesh, compiler_params=cparams,
        scratch_types=[
            pltpu.VMEM_SHARED((NPAD, D), f32),
            pltpu.VMEM((8, CH), jnp.int32),
            pltpu.VMEM((8, CH), jnp.int32),
            pltpu.VMEM((CH, D), f32),
            pltpu.VMEM((CH, D), f32),
            pltpu.VMEM((8, CH), f32),
            pltpu.VMEM((8, CH), f32),
            pltpu.SemaphoreType.DMA,
            pltpu.SemaphoreType.DMA,
            pltpu.SemaphoreType.DMA,
            pltpu.SemaphoreType.DMA,
        ],
    )(rowi, coli, pdeg, embf)

    p2, o1h, _xs2 = pl.kernel(
        _k3_body,
        out_type=(jax.ShapeDtypeStruct((NC, NPAD, D), f32),
                  jax.ShapeDtypeStruct((NC * NPAD, D), f32),
                  jax.ShapeDtypeStruct((NC * NPAD, D), f32)),
        mesh=mesh, compiler_params=cparams,
        scratch_types=[
            pltpu.VMEM_SHARED((NPAD, D), f32),
            pltpu.VMEM((8, CH), jnp.int32),
            pltpu.VMEM((8, CH), jnp.int32),
            pltpu.VMEM((CH, D), f32),
            pltpu.VMEM((CH, D), f32),
            pltpu.VMEM((8, CH), f32),
            pltpu.SemaphoreType.DMA,
            pltpu.SemaphoreType.DMA,
            pltpu.SemaphoreType.DMA,
            pltpu.SemaphoreType.DMA,
        ],
    )(rowi, coli, p1, dish)

    pdots, _out = pl.kernel(
        _k4_body,
        out_type=(jax.ShapeDtypeStruct((NC, NS, LROWS, LCH), f32),
                  jax.ShapeDtypeStruct((NC * NPAD, D), f32)),
        mesh=mesh, compiler_params=cparams,
        scratch_types=[
            pltpu.VMEM((LROWS, LCH), jnp.int32),
            pltpu.VMEM((LROWS, LCH), jnp.int32),
            pltpu.VMEM((CH, D), f32),
            pltpu.VMEM((CH, D), f32),
            pltpu.VMEM((8, CH), f32),
            pltpu.VMEM((LROWS, LCH), f32),
            pltpu.SemaphoreType.DMA,
            pltpu.SemaphoreType.DMA,
        ],
    )(srci, dsti, p2, dish, o1h, embf)
    return pdots[:, :, :LNCH, :].reshape(LE)


def kernel(edge_index, edge_label_index, emb):
    return _negative_prop_sc(edge_index, edge_label_index, emb)


# speedup vs baseline: 15.1275x; 1.0566x over previous
import jax
import jax.numpy as jnp
from jax import lax
from jax.experimental import pallas as pl
from jax.experimental.pallas import tpu as pltpu
from jax.experimental.pallas import tpu_sc as plsc

N = 10000
NPAD = 10240
D = 128
E = 320000
LE = 8192
NS = 16
NC = 2
CH = 128
EPT = E // (NC * NS)
NCHUNK = 80
ET = NCHUNK * CH
NT = NPAD // NS
NTC = NT // CH
LT = LE // (NC * NS)
LCH = 128
LNCH = LT // LCH
LROWS = 8
ALPHA = 1.0 / 3.0

_BCAST_DNUMS = jax.lax.GatherDimensionNumbers(
    offset_dims=(), collapsed_slice_dims=(0,), start_index_map=(0,))


def _bcast(vec16, lane):
    idx = jnp.full((16, 1), lane, jnp.int32)
    return jax.lax.gather(vec16, idx, _BCAST_DNUMS, slice_sizes=(1,),
                          mode=jax.lax.GatherScatterMode.PROMISE_IN_BOUNDS)


def _rsqrt16(d):
    i = lax.bitcast_convert_type(d, jnp.int32)
    i = jnp.int32(0x5F3759DF) - (i >> 1)
    y = lax.bitcast_convert_type(i, jnp.float32)
    for _ in range(3):
        y = y * (1.5 - 0.5 * d * y * y)
    return y


def _fill(buf, v16):
    def _frow(r, carry):
        for q in range(D // 16):
            buf[r, pl.ds(q * 16, 16)] = v16
        return carry
    lax.fori_loop(0, CH, _frow, 0)


def _zero_acc_slice(acc_sh, zbuf, nbase):
    def _z(t, carry):
        pltpu.sync_copy(zbuf, acc_sh.at[pl.ds(nbase + t * CH, CH), :])
        return carry
    lax.fori_loop(0, NTC, _z, 0)


def _edge_pass(c, s, rowi, coli, xsrc, acc_sh, rring, cring, gbuf, nbuf,
               semg0, semg1, sems0, sems1):
    bufs = (gbuf, nbuf)
    gsems = (semg0, semg1)
    ssems = (sems0, sems1)

    def _grp(g, carry):
        pltpu.sync_copy(rowi.at[c, s, pl.ds(g * 8, 8), :], rring)
        pltpu.sync_copy(coli.at[c, s, pl.ds(g * 8, 8), :], cring)
        pltpu.async_copy(xsrc.at[rring.at[0]], gbuf, semg0)
        pltpu.async_copy(xsrc.at[rring.at[1]], nbuf, semg1)
        for k in range(8):
            b = bufs[k % 2]
            pltpu.make_async_copy(
                xsrc.at[rring.at[k]], b, gsems[k % 2]).wait()
            pltpu.async_copy(
                b, acc_sh.at[cring.at[k]], ssems[k % 2], add=True)
            if k + 2 < 8:
                pltpu.make_async_copy(
                    b, acc_sh.at[cring.at[k]], ssems[k % 2]).wait()
                pltpu.async_copy(
                    xsrc.at[rring.at[k + 2]], b, gsems[k % 2])
        for k in (6, 7):
            pltpu.make_async_copy(
                bufs[k % 2], acc_sh.at[cring.at[k]], ssems[k % 2]).wait()
        return carry
    lax.fori_loop(0, NCHUNK // 8, _grp, 0)
    plsc.subcore_barrier()


def _k1_body(coli, pdeg, acc_sh, cring, gbuf, nbuf, pbuf, semg0):
    c = lax.axis_index("c")
    s = lax.axis_index("s")
    nbase = s * NT
    z16 = jnp.zeros((16,), jnp.float32)
    o16 = jnp.ones((16,), jnp.float32)
    _fill(nbuf, z16)
    _fill(gbuf, o16)
    _zero_acc_slice(acc_sh, nbuf, nbase)
    plsc.subcore_barrier()

    def _hist(g, carry):
        pltpu.sync_copy(coli.at[c, s, pl.ds(g * 8, 8), :], cring)

        def _h8(k, carry2):
            pltpu.async_copy(gbuf, acc_sh.at[cring.at[k]], semg0, add=True)
            return carry2
        lax.fori_loop(0, 8, _h8, 0)

        def _d8(k, carry2):
            pltpu.make_async_copy(gbuf, acc_sh.at[cring.at[k]], semg0).wait()
            return carry2
        lax.fori_loop(0, 8, _d8, 0)
        return carry
    lax.fori_loop(0, NCHUNK // 8, _hist, 0)
    plsc.subcore_barrier()

    iota16 = lax.iota(jnp.int32, 16)
    zc16 = jnp.zeros((16,), jnp.int32)

    def _ext(t, carry):
        pltpu.sync_copy(acc_sh.at[pl.ds(nbase + t * CH, CH), :], nbuf)

        def _e16(k, carry2):
            rows = k * 16 + iota16
            dv = plsc.load_gather(nbuf, [rows, zc16])
            pbuf[t, pl.ds(k * 16, 16)] = dv
            return carry2
        lax.fori_loop(0, CH // 16, _e16, 0)
        return carry
    lax.fori_loop(0, NTC, _ext, 0)
    pltpu.sync_copy(pbuf, pdeg.at[c, s])


def _k2_body(rowi, coli, pdeg, embf, p1, dish, xsh,
             acc_sh, rring, cring, gbuf, nbuf, dbuf, ebuf,
             semg0, semg1, sems0, sems1):
    c = lax.axis_index("c")
    s = lax.axis_index("s")
    nbase = s * NT
    fbase = c * NPAD
    z16 = jnp.zeros((16,), jnp.float32)
    _fill(nbuf, z16)
    _zero_acc_slice(acc_sh, nbuf, nbase)

    pltpu.sync_copy(pdeg.at[0, s], dbuf)
    pltpu.sync_copy(pdeg.at[1, s], ebuf)

    def _rs(t, carry):
        for k in range(CH // 16):
            sl = pl.ds(k * 16, 16)
            dv = dbuf[t, sl] + ebuf[t, sl]
            y = _rsqrt16(dv)
            dbuf[t, sl] = jnp.where(dv > 0.5, y, 0.0)
        return carry
    lax.fori_loop(0, NTC, _rs, 0)
    pltpu.sync_copy(dbuf, dish.at[c, s])

    bufs = (nbuf, gbuf)
    gsems = (semg0, semg1)
    wsems = (sems0, sems1)
    prev_w = [None, None]
    pltpu.async_copy(embf.at[pl.ds(nbase, CH), :], nbuf, semg0)
    for t in range(NTC):
        b = bufs[t % 2]
        n0 = nbase + t * CH
        pltpu.make_async_copy(
            embf.at[pl.ds(n0, CH), :], b, gsems[t % 2]).wait()
        if t + 1 < NTC:
            nb = bufs[(t + 1) % 2]
            if prev_w[(t + 1) % 2] is not None:
                prev_w[(t + 1) % 2].wait()
            pltpu.async_copy(embf.at[pl.ds(n0 + CH, CH), :], nb,
                             gsems[(t + 1) % 2])

        def _grp(g, carry2, t=t, b=b):
            dv = dbuf[t, pl.ds(g * 16, 16)]
            for j in range(16):
                r = g * 16 + j
                db = _bcast(dv, j)
                for q in range(D // 16):
                    sl = pl.ds(q * 16, 16)
                    b[r, sl] = b[r, sl] * db
            return carry2
        lax.fori_loop(0, CH // 16, _grp, 0)
        pltpu.async_copy(b, xsh.at[pl.ds(fbase + n0, CH), :], wsems[t % 2])
        prev_w[t % 2] = pltpu.make_async_copy(
            b, xsh.at[pl.ds(fbase + n0, CH), :], wsems[t % 2])
    for w in prev_w:
        if w is not None:
            w.wait()
    plsc.subcore_barrier()

    _edge_pass(c, s, rowi, coli, xsh, acc_sh, rring, cring, gbuf, nbuf,
               semg0, semg1, sems0, sems1)

    def _wr(t, carry):
        n0 = nbase + t * CH
        pltpu.sync_copy(acc_sh.at[pl.ds(n0, CH), :], nbuf)
        pltpu.sync_copy(nbuf, p1.at[c, pl.ds(n0, CH), :])
        return carry
    lax.fori_loop(0, NTC, _wr, 0)


def _k3_body(rowi, coli, p1, dish, p2, o1h, xs2h,
             acc_sh, rring, cring, gbuf, nbuf, dbuf,
             semg0, semg1, sems0, sems1):
    c = lax.axis_index("c")
    s = lax.axis_index("s")
    nbase = s * NT
    fbase = c * NPAD
    z16 = jnp.zeros((16,), jnp.float32)
    _fill(nbuf, z16)
    _zero_acc_slice(acc_sh, nbuf, nbase)
    pltpu.sync_copy(dish.at[c, s], dbuf)

    for t in range(NTC):
        n0 = nbase + t * CH
        pltpu.async_copy(p1.at[0, pl.ds(n0, CH), :], nbuf, semg0)
        pltpu.async_copy(p1.at[1, pl.ds(n0, CH), :], gbuf, semg1)
        pltpu.make_async_copy(p1.at[0, pl.ds(n0, CH), :], nbuf, semg0).wait()
        pltpu.make_async_copy(p1.at[1, pl.ds(n0, CH), :], gbuf, semg1).wait()

        def _grp(g, carry2, t=t):
            dv = dbuf[t, pl.ds(g * 16, 16)]
            for j in range(16):
                r = g * 16 + j
                db = _bcast(dv, j)
                for q in range(D // 16):
                    sl = pl.ds(q * 16, 16)
                    u = (nbuf[r, sl] + gbuf[r, sl]) * db
                    nbuf[r, sl] = u * db
                    gbuf[r, sl] = u * ALPHA
            return carry2
        lax.fori_loop(0, CH // 16, _grp, 0)
        pltpu.async_copy(nbuf, xs2h.at[pl.ds(fbase + n0, CH), :], sems0)
        pltpu.async_copy(gbuf, o1h.at[pl.ds(fbase + n0, CH), :], sems1)
        pltpu.make_async_copy(
            nbuf, xs2h.at[pl.ds(fbase + n0, CH), :], sems0).wait()
        pltpu.make_async_copy(
            gbuf, o1h.at[pl.ds(fbase + n0, CH), :], sems1).wait()
    plsc.subcore_barrier()

    _edge_pass(c, s, rowi, coli, xs2h, acc_sh, rring, cring, gbuf, nbuf,
               semg0, semg1, sems0, sems1)

    def _wr(t, carry):
        n0 = nbase + t * CH
        pltpu.sync_copy(acc_sh.at[pl.ds(n0, CH), :], nbuf)
        pltpu.sync_copy(nbuf, p2.at[c, pl.ds(n0, CH), :])
        return carry
    lax.fori_loop(0, NTC, _wr, 0)


def _k4_body(srci, dsti, p2, dish, o1h, embf, pdots, outh,
             sidx, didx, gbuf, nbuf, obuf, ebuf, wbuf, dbuf, dotsb,
             semg0, semg1, semg2, semg3, semw):
    c = lax.axis_index("c")
    s = lax.axis_index("s")
    nbase = s * NT
    fbase = c * NPAD
    z16 = jnp.zeros((16,), jnp.float32)
    pltpu.sync_copy(dish.at[c, s], dbuf)
    pltpu.sync_copy(srci.at[c, s], sidx)
    pltpu.sync_copy(dsti.at[c, s], didx)

    def _zdots(t, carry):
        for g in range(LCH // 16):
            dotsb[t, pl.ds(g * 16, 16)] = z16
        return carry
    lax.fori_loop(0, LROWS, _zdots, 0)

    def _issue_loads(t):
        n0 = nbase + t * CH
        pltpu.async_copy(p2.at[0, pl.ds(n0, CH), :], nbuf, semg0)
        pltpu.async_copy(p2.at[1, pl.ds(n0, CH), :], gbuf, semg1)
        pltpu.async_copy(o1h.at[pl.ds(fbase + n0, CH), :], obuf, semg2)
        pltpu.async_copy(embf.at[pl.ds(n0, CH), :], ebuf, semg3)

    def _wait_loads(t):
        n0 = nbase + t * CH
        pltpu.make_async_copy(
            p2.at[0, pl.ds(n0, CH), :], nbuf, semg0).wait()
        pltpu.make_async_copy(
            p2.at[1, pl.ds(n0, CH), :], gbuf, semg1).wait()
        pltpu.make_async_copy(
            o1h.at[pl.ds(fbase + n0, CH), :], obuf, semg2).wait()
        pltpu.make_async_copy(
            embf.at[pl.ds(n0, CH), :], ebuf, semg3).wait()

    prev_w = None
    _issue_loads(0)
    for t in range(NTC):
        n0 = nbase + t * CH
        _wait_loads(t)
        if prev_w is not None:
            prev_w.wait()

        def _g1(g, carry2, t=t):
            dv = dbuf[t, pl.ds(g * 16, 16)]
            for j in range(16):
                r = g * 16 + j
                db = _bcast(dv, j)
                for q in range(D // 16):
                    sl = pl.ds(q * 16, 16)
                    wbuf[r, sl] = ((nbuf[r, sl] + gbuf[r, sl]) * db
                                   + ebuf[r, sl]) * ALPHA + obuf[r, sl]
            return carry2
        lax.fori_loop(0, CH // 16, _g1, 0)
        if t + 1 < NTC:
            _issue_loads(t + 1)
        pltpu.async_copy(wbuf, outh.at[pl.ds(fbase + n0, CH), :], semw)
        prev_w = pltpu.make_async_copy(
            wbuf, outh.at[pl.ds(fbase + n0, CH), :], semw)
    prev_w.wait()
    plsc.subcore_barrier()

    iota16 = lax.iota(jnp.int32, 16)

    def _lab(t, carry):
        gs = pltpu.async_copy(outh.at[sidx.at[t]], gbuf, semg0)
        gd = pltpu.async_copy(outh.at[didx.at[t]], nbuf, semg1)
        gs.wait()
        gd.wait()

        def _grp(g, carry2):
            rows = g * 16 + iota16
            acc = z16
            for dd in range(D):
                cd = jnp.full((16,), dd, jnp.int32)
                sv = plsc.load_gather(gbuf, [rows, cd])
                dv = plsc.load_gather(nbuf, [rows, cd])
                acc = acc + sv * dv
            dotsb[t, pl.ds(g * 16, 16)] = acc
            return carry2
        lax.fori_loop(0, LCH // 16, _grp, 0)
        return carry
    lax.fori_loop(0, LNCH, _lab, 0)
    pltpu.sync_copy(dotsb, pdots.at[c, s])


@jax.jit
def _negative_prop_sc(edge_index, edge_label_index, emb):
    ei32 = edge_index.astype(jnp.int32).reshape(2, NC, NS, EPT)
    padv = N + jnp.arange(ET - EPT, dtype=jnp.int32) % (NPAD - N)
    padb = jnp.broadcast_to(padv, (2, NC, NS, ET - EPT))
    eall = jnp.concatenate([ei32, padb], axis=3).reshape(
        2, NC, NS, NCHUNK, CH)
    off = jnp.array([0, NPAD], jnp.int32).reshape(NC, 1, 1, 1)
    rowi = eall[0] + off
    coli = eall[1]
    eli32 = edge_label_index.astype(jnp.int32).reshape(2, NC, NS, LNCH, LCH)
    loff = jnp.array([0, NPAD], jnp.int32).reshape(NC, 1, 1, 1)
    lpad = jnp.full((NC, NS, LROWS - LNCH, LCH), N, jnp.int32) + loff
    srci = jnp.concatenate([eli32[0] + loff, lpad], axis=2)
    dsti = jnp.concatenate([eli32[1] + loff, lpad], axis=2)
    embf = jnp.zeros((NPAD, D), jnp.float32).at[:N].set(emb)

    mesh = plsc.VectorSubcoreMesh(core_axis_name="c", subcore_axis_name="s")
    cparams = pltpu.CompilerParams(needs_layout_passes=False)
    f32 = jnp.float32

    pdeg = pl.kernel(
        _k1_body,
        out_type=jax.ShapeDtypeStruct((NC, NS, 8, CH), f32),
        mesh=mesh, compiler_params=cparams,
        scratch_types=[
            pltpu.VMEM_SHARED((NPAD, D), f32),
            pltpu.VMEM((8, CH), jnp.int32),
            pltpu.VMEM((CH, D), f32),
            pltpu.VMEM((CH, D), f32),
            pltpu.VMEM((8, CH), f32),
            pltpu.SemaphoreType.DMA,
        ],
    )(coli)

    p1, dish, _xs = pl.kernel(
        _k2_body,
        out_type=(jax.ShapeDtypeStruct((NC, NPAD, D), f32),
                  jax.ShapeDtypeStruct((NC, NS, 8, CH), f32),
                  jax.ShapeDtypeStruct((NC * NPAD, D), f32)),
        mesh=mesh, compiler_params=cparams,
        scratch_types=[
            pltpu.VMEM_SHARED((NPAD, D), f32),
            pltpu.VMEM((8, CH), jnp.int32),
            pltpu.VMEM((8, CH), jnp.int32),
            pltpu.VMEM((CH, D), f32),
            pltpu.VMEM((CH, D), f32),
            pltpu.VMEM((8, CH), f32),
            pltpu.VMEM((8, CH), f32),
            pltpu.SemaphoreType.DMA,
            pltpu.SemaphoreType.DMA,
            pltpu.SemaphoreType.DMA,
            pltpu.SemaphoreType.DMA,
        ],
    )(rowi, coli, pdeg, embf)

    p2, o1h, _xs2 = pl.kernel(
        _k3_body,
        out_type=(jax.ShapeDtypeStruct((NC, NPAD, D), f32),
                  jax.ShapeDtypeStruct((NC * NPAD, D), f32),
                  jax.ShapeDtypeStruct((NC * NPAD, D), f32)),
        mesh=mesh, compiler_params=cparams,
        scratch_types=[
            pltpu.VMEM_SHARED((NPAD, D), f32),
            pltpu.VMEM((8, CH), jnp.int32),
            pltpu.VMEM((8, CH), jnp.int32),
            pltpu.VMEM((CH, D), f32),
            pltpu.VMEM((CH, D), f32),
            pltpu.VMEM((8, CH), f32),
            pltpu.SemaphoreType.DMA,
            pltpu.SemaphoreType.DMA,
            pltpu.SemaphoreType.DMA,
            pltpu.SemaphoreType.DMA,
        ],
    )(rowi, coli, p1, dish)

    pdots, _out = pl.kernel(
        _k4_body,
        out_type=(jax.ShapeDtypeStruct((NC, NS, LROWS, LCH), f32),
                  jax.ShapeDtypeStruct((NC * NPAD, D), f32)),
        mesh=mesh, compiler_params=cparams,
        scratch_types=[
            pltpu.VMEM((LROWS, LCH), jnp.int32),
            pltpu.VMEM((LROWS, LCH), jnp.int32),
            pltpu.VMEM((CH, D), f32),
            pltpu.VMEM((CH, D), f32),
            pltpu.VMEM((CH, D), f32),
            pltpu.VMEM((CH, D), f32),
            pltpu.VMEM((CH, D), f32),
            pltpu.VMEM((8, CH), f32),
            pltpu.VMEM((LROWS, LCH), f32),
            pltpu.SemaphoreType.DMA,
            pltpu.SemaphoreType.DMA,
            pltpu.SemaphoreType.DMA,
            pltpu.SemaphoreType.DMA,
            pltpu.SemaphoreType.DMA,
        ],
    )(srci, dsti, p2, dish, o1h, embf)
    return pdots[:, :, :LNCH, :].reshape(LE)


def kernel(edge_index, edge_label_index, emb):
    return _negative_prop_sc(edge_index, edge_label_index, emb)
